# baseline (speedup 1.0000x reference)
import functools

import jax
import jax.numpy as jnp
from jax import lax
from jax.experimental import pallas as pl
from jax.experimental.pallas import tpu as pltpu
import jax.experimental.pallas.tpu_sc as plsc

NN = 10000
NP = 10240
NE = 160000
NEA = 76800
NEB = NE - NEA
DN = 42
DH = 128
DP = 128
OP = 64
FW = DN * OP
NSTEPS = 6

NC, NS = 2, 16
NW = NC * NS
GCH = 200
SCH = 184


def _schunks(epw):
    full = (epw - 8) // SCH
    rem = epw - full * SCH
    return tuple((i * SCH, SCH) for i in range(full)) + ((full * SCH, rem),)


_F32 = jnp.float32


def _lin0_body(nf_ref, w_ref, b_ref, out_ref):
    y = jnp.dot(nf_ref[...], w_ref[...], preferred_element_type=_F32)
    out_ref[...] = jnp.maximum(y + b_ref[...], 0.0)


def _lin0(nf_p, w0p, b0p):
    return pl.pallas_call(
        _lin0_body,
        out_shape=jax.ShapeDtypeStruct((NP, DP), _F32),
    )(nf_p, w0p, b0p)


EB1 = 6400


def _hidprep_body(ef_ref, we1_ref, be1_ref, hid_ref):
    hid = jnp.dot(ef_ref[...], we1_ref[...], preferred_element_type=_F32)
    hid_ref[...] = jnp.maximum(hid + be1_ref[...], 0.0
                               ).astype(jnp.bfloat16)


def _hidprep(ef_p, we1p, be1r, ne):
    grid = ne // EB1
    return pl.pallas_call(
        _hidprep_body,
        grid=(grid,),
        in_specs=[
            pl.BlockSpec((EB1, 16), lambda i: (i, 0)),
            pl.BlockSpec((16, DH), lambda i: (0, 0)),
            pl.BlockSpec((1, DH), lambda i: (0, 0)),
        ],
        out_specs=pl.BlockSpec((EB1, DH), lambda i: (i, 0)),
        out_shape=jax.ShapeDtypeStruct((ne, DH), jnp.bfloat16),
        compiler_params=pltpu.CompilerParams(
            dimension_semantics=("parallel",)),
    )(ef_p, we1p, be1r)


EB3 = 800


def _msg_body(hid_ref, h_ref, bsel_ref, we2_ref, be2_ref, msg_ref):
    h48 = h_ref[:, :48].astype(jnp.bfloat16)
    w = jnp.dot(hid_ref[...], we2_ref[...], preferred_element_type=_F32)
    hexp = jnp.dot(h48, bsel_ref[...], preferred_element_type=_F32)
    acc = jnp.zeros((EB3, 128), _F32)
    for j in range(FW // 128):
        sl = slice(128 * j, 128 * (j + 1))
        acc = acc + w[:, sl] * hexp[:, sl]
    fold = (acc[:, :OP] + acc[:, OP:]
            + jnp.dot(h48, be2_ref[...], preferred_element_type=_F32))
    msg_ref[...] = jnp.concatenate(
        [fold, jnp.zeros((EB3, DP - OP), _F32)], axis=1)


def _msg(hid, h, bsel, we2b, be2m, ne):
    grid = ne // EB3
    return pl.pallas_call(
        _msg_body,
        grid=(grid,),
        in_specs=[
            pl.BlockSpec((EB3, DH), lambda i: (i, 0)),
            pl.BlockSpec((EB3, DP), lambda i: (i, 0)),
            pl.BlockSpec((48, FW), lambda i: (0, 0)),
            pl.BlockSpec((DH, FW), lambda i: (0, 0)),
            pl.BlockSpec((48, OP), lambda i: (0, 0)),
        ],
        out_specs=pl.BlockSpec((EB3, DP), lambda i: (i, 0)),
        out_shape=jax.ShapeDtypeStruct((ne, DP), _F32),
        compiler_params=pltpu.CompilerParams(
            dimension_semantics=("parallel",)),
    )(hid, h, bsel, we2b, be2m)


NBUF = 4


def _sc_gather(table, src, ne):
    epw = ne // NW
    nch = epw // GCH
    mesh = plsc.VectorSubcoreMesh(core_axis_name="c", subcore_axis_name="s")

    @functools.partial(
        pl.kernel,
        out_type=jax.ShapeDtypeStruct((ne, DP), _F32),
        mesh=mesh,
        scratch_types=(
            [pltpu.VMEM((GCH,), jnp.int32)] * NBUF
            + [pltpu.VMEM((GCH, DP), _F32)] * NBUF
            + [pltpu.SemaphoreType.DMA((NBUF,))] * 3
        ),
    )
    def k(table_hbm, src_hbm, out_hbm, i0, i1, i2, i3, r0, r1, r2, r3,
          isem, gsem, osem):
        idx_v = (i0, i1, i2, i3)
        rows_v = (r0, r1, r2, r3)
        wid = lax.axis_index("s") * NC + lax.axis_index("c")
        base = wid * epw

        def icp(ci):
            b = ci % NBUF
            return pltpu.make_async_copy(
                src_hbm.at[pl.ds(base + ci * GCH, GCH)], idx_v[b],
                isem.at[b])

        def gat(ci):
            b = ci % NBUF
            return pltpu.make_async_copy(
                table_hbm.at[idx_v[b]], rows_v[b], gsem.at[b])

        def wb(ci):
            b = ci % NBUF
            return pltpu.make_async_copy(
                rows_v[b], out_hbm.at[pl.ds(base + ci * GCH, GCH)],
                osem.at[b])

        icp(0).start()
        icp(1).start()
        for ci in range(nch):
            icp(ci).wait()
            gat(ci).start()
            if ci >= 1:
                gat(ci - 1).wait()
                wb(ci - 1).start()
            if ci >= 2:
                wb(ci - 2).wait()
            if ci + 2 < nch:
                icp(ci + 2).start()
        gat(nch - 1).wait()
        wb(nch - 1).start()
        wb(nch - 2).wait()
        wb(nch - 1).wait()

    return k(table, src)


def _sc_scatter(msg, dst, zeros_t, ne):
    epw = ne // NW
    chunks = _schunks(epw)
    nsch = len(chunks)
    mesh = plsc.VectorSubcoreMesh(core_axis_name="c", subcore_axis_name="s")
    ZR = NP // NS

    @functools.partial(
        pl.kernel,
        out_type=jax.ShapeDtypeStruct((NC, NP, DP), _F32),
        mesh=mesh,
        scratch_types=(
            [pltpu.VMEM((SCH,), jnp.int32)] * 2
            + [pltpu.VMEM((SCH, DP), _F32)] * 2
            + [pltpu.VMEM_SHARED((NP, DP), _F32)]
            + [pltpu.SemaphoreType.DMA((2,))] * 2
        ),
    )
    def k(msg_hbm, dst_hbm, zeros_hbm, agg_hbm, i0, i1, r0, r1,
          table_sh, lsem, ssem):
        idx_v = (i0, i1)
        rows_v = (r0, r1)
        c = lax.axis_index("c")
        s = lax.axis_index("s")
        pltpu.sync_copy(zeros_hbm.at[pl.ds(s * ZR, ZR)],
                        table_sh.at[pl.ds(s * ZR, ZR)])
        plsc.subcore_barrier()
        base = (s * NC + c) * epw

        def icp(ci):
            off, n, b = chunks[ci][0], chunks[ci][1], ci % 2
            return pltpu.make_async_copy(
                dst_hbm.at[pl.ds(base + off, n)],
                idx_v[b].at[pl.ds(0, n)], lsem.at[b])

        def mcp(ci):
            off, n, b = chunks[ci][0], chunks[ci][1], ci % 2
            return pltpu.make_async_copy(
                msg_hbm.at[pl.ds(base + off, n)],
                rows_v[b].at[pl.ds(0, n)], lsem.at[b])

        def sca(ci, start):
            n, b = chunks[ci][1], ci % 2
            d = pltpu.make_async_copy(
                rows_v[b].at[pl.ds(0, n)],
                table_sh.at[idx_v[b].at[pl.ds(0, n)]], ssem.at[b])
            if start:
                pltpu.async_copy(
                    rows_v[b].at[pl.ds(0, n)],
                    table_sh.at[idx_v[b].at[pl.ds(0, n)]], ssem.at[b],
                    add=True)
            else:
                d.wait()

        icp(0).start()
        mcp(0).start()
        for ci in range(nsch):
            icp(ci).wait()
            mcp(ci).wait()
            sca(ci, True)
            if ci >= 1:
                sca(ci - 1, False)
            if ci + 1 < nsch:
                icp(ci + 1).start()
                mcp(ci + 1).start()
        sca(nsch - 1, False)
        plsc.subcore_barrier()
        pltpu.sync_copy(table_sh.at[pl.ds(s * ZR, ZR)],
                        agg_hbm.at[c, pl.ds(s * ZR, ZR)])

    return k(msg, dst, zeros_t)


def _upd_body(aggpa_ref, aggpb_ref, out_ref, cb_ref, wm1_ref, wm2_ref,
              bm_ref, new_ref):
    agg = (aggpa_ref[0] + aggpa_ref[1]) + (aggpb_ref[0] + aggpb_ref[1])
    o = out_ref[...]
    m = jnp.maximum(agg + o + cb_ref[...], 0.0)
    y = (jnp.dot(m, wm1_ref[...], preferred_element_type=_F32)
         + jnp.dot(o, wm2_ref[...], preferred_element_type=_F32))
    new_ref[...] = y + bm_ref[...]


def _upd_res_body(aggpa_ref, aggpb_ref, out_ref, cb_ref, wm1_ref, wm2_ref,
                  bm_ref, nf_ref, new_ref):
    agg = (aggpa_ref[0] + aggpa_ref[1]) + (aggpb_ref[0] + aggpb_ref[1])
    o = out_ref[...]
    m = jnp.maximum(agg + o + cb_ref[...], 0.0)
    y = (jnp.dot(m, wm1_ref[...], preferred_element_type=_F32)
         + jnp.dot(o, wm2_ref[...], preferred_element_type=_F32))
    new_ref[...] = y + bm_ref[...] + nf_ref[...]


def _upd(aggpa, aggpb, out, cbp, wm1p, wm2p, bmp):
    return pl.pallas_call(
        _upd_body,
        out_shape=jax.ShapeDtypeStruct((NP, DP), _F32),
    )(aggpa, aggpb, out, cbp, wm1p, wm2p, bmp)


def _upd_res(aggpa, aggpb, out, cbp, wm1p, wm2p, bmp, nf_p):
    return pl.pallas_call(
        _upd_res_body,
        out_shape=jax.ShapeDtypeStruct((NP, DP), _F32),
    )(aggpa, aggpb, out, cbp, wm1p, wm2p, bmp, nf_p)


def _pad2(a, rows, cols):
    return jnp.pad(a, ((0, rows - a.shape[0]), (0, cols - a.shape[1])))


def kernel(n_feat, e_feat, edge_index, W0, b0, We1, be1, We2, be2,
           conv_bias, Wm, bm):
    src_a, src_b = edge_index[0, :NEA], edge_index[0, NEA:]
    dst_a, dst_b = edge_index[1, :NEA], edge_index[1, NEA:]

    nf_p = _pad2(n_feat, NP, DP)
    ef_p = _pad2(e_feat, NE, 16)
    w0p = _pad2(W0, DP, DP)
    b0p = _pad2(b0[None, :], 1, DP)
    we1p = _pad2(We1, 16, DH)
    be1r = be1[None, :]
    we2b = jnp.pad(We2.reshape(DH, DN, DN), ((0, 0), (0, 0), (0, OP - DN))
                   ).reshape(DH, FW).astype(jnp.bfloat16)
    be2m = jnp.pad(be2.reshape(DN, DN), ((0, 48 - DN), (0, OP - DN))
                   ).astype(jnp.bfloat16)
    cbp = _pad2(conv_bias[None, :], 1, DP)
    wm1p = _pad2(Wm[:DN], DP, DP)
    wm2p = _pad2(Wm[DN:], DP, DP)
    bmp = _pad2(bm[None, :], 1, DP)
    zeros_t = jnp.zeros((NP, DP), _F32)
    bsel = (jnp.arange(FW)[None, :] // OP ==
            jnp.arange(48)[:, None]).astype(jnp.bfloat16)

    out = _lin0(nf_p, w0p, b0p)
    hid_a = _hidprep(ef_p[:NEA], we1p, be1r, NEA)
    hid_b = _hidprep(ef_p[NEA:], we1p, be1r, NEB)

    for step in range(NSTEPS):
        h_a = _sc_gather(out, src_a, NEA)
        msg_a = _msg(hid_a, h_a, bsel, we2b, be2m, NEA)
        h_b = _sc_gather(out, src_b, NEB)
        msg_b = _msg(hid_b, h_b, bsel, we2b, be2m, NEB)
        aggp_a = _sc_scatter(msg_a, dst_a, zeros_t, NEA)
        aggp_b = _sc_scatter(msg_b, dst_b, zeros_t, NEB)
        if step < NSTEPS - 1:
            out = _upd(aggp_a, aggp_b, out, cbp, wm1p, wm2p, bmp)
        else:
            out = _upd_res(aggp_a, aggp_b, out, cbp, wm1p, wm2p, bmp, nf_p)

    return out[:NN, :DN]


# speedup vs baseline: 1.1799x; 1.1799x over previous
import functools

import jax
import jax.numpy as jnp
from jax import lax
from jax.experimental import pallas as pl
from jax.experimental.pallas import tpu as pltpu
import jax.experimental.pallas.tpu_sc as plsc

NN = 10000
NP = 10240
NE = 160000
NEA = 76800
NEB = NE - NEA
DN = 42
DH = 128
DP = 128
OP = 64
FW = DN * OP
NSTEPS = 6

NC, NS = 2, 16
NW = NC * NS
GCH = 200
SCH = 184


def _schunks(epw):
    full = (epw - 8) // SCH
    rem = epw - full * SCH
    return tuple((i * SCH, SCH) for i in range(full)) + ((full * SCH, rem),)


_F32 = jnp.float32


def _lin0_body(nf_ref, w_ref, b_ref, out_ref):
    y = jnp.dot(nf_ref[...], w_ref[...], preferred_element_type=_F32)
    out_ref[...] = jnp.maximum(y + b_ref[...], 0.0)


def _lin0(nf_p, w0p, b0p):
    return pl.pallas_call(
        _lin0_body,
        out_shape=jax.ShapeDtypeStruct((NP, DP), _F32),
    )(nf_p, w0p, b0p)


EB1 = 1600


def _wprep_body(ef_ref, we1_ref, be1_ref, we2_ref, be2_ref, ew_ref):
    hid = jnp.dot(ef_ref[...], we1_ref[...], preferred_element_type=_F32)
    hid = jnp.maximum(hid + be1_ref[...], 0.0)
    w = jnp.dot(hid.astype(jnp.bfloat16), we2_ref[...],
                preferred_element_type=_F32)
    ew_ref[...] = (w + be2_ref[...]).astype(jnp.bfloat16)


def _wprep(ef_p, we1p, be1r, we2b, be2b, ne):
    grid = ne // EB1
    return pl.pallas_call(
        _wprep_body,
        grid=(grid,),
        in_specs=[
            pl.BlockSpec((EB1, 16), lambda i: (i, 0)),
            pl.BlockSpec((16, DH), lambda i: (0, 0)),
            pl.BlockSpec((1, DH), lambda i: (0, 0)),
            pl.BlockSpec((DH, FW), lambda i: (0, 0)),
            pl.BlockSpec((1, FW), lambda i: (0, 0)),
        ],
        out_specs=pl.BlockSpec((EB1, FW), lambda i: (i, 0)),
        out_shape=jax.ShapeDtypeStruct((ne, FW), jnp.bfloat16),
        compiler_params=pltpu.CompilerParams(
            dimension_semantics=("parallel",)),
    )(ef_p, we1p, be1r, we2b, be2b)


EB3 = 1600


def _msg_body(w_ref, h_ref, bsel_ref, msg_ref):
    hexp = jnp.dot(h_ref[:, :48].astype(jnp.bfloat16), bsel_ref[...],
                   preferred_element_type=_F32)
    acc = jnp.zeros((EB3, 128), _F32)
    for j in range(FW // 128):
        sl = slice(128 * j, 128 * (j + 1))
        acc = acc + w_ref[:, sl].astype(_F32) * hexp[:, sl]
    fold = acc[:, :OP] + acc[:, OP:]
    msg_ref[...] = jnp.concatenate(
        [fold, jnp.zeros((EB3, DP - OP), _F32)], axis=1)


def _msg(ew, h, bsel, ne):
    grid = ne // EB3
    return pl.pallas_call(
        _msg_body,
        grid=(grid,),
        in_specs=[
            pl.BlockSpec((EB3, FW), lambda i: (i, 0)),
            pl.BlockSpec((EB3, DP), lambda i: (i, 0)),
            pl.BlockSpec((48, FW), lambda i: (0, 0)),
        ],
        out_specs=pl.BlockSpec((EB3, DP), lambda i: (i, 0)),
        out_shape=jax.ShapeDtypeStruct((ne, DP), _F32),
        compiler_params=pltpu.CompilerParams(
            dimension_semantics=("parallel",)),
    )(ew, h, bsel)


NBUF = 4


def _sc_gather(table, src, ne):
    epw = ne // NW
    nch = epw // GCH
    mesh = plsc.VectorSubcoreMesh(core_axis_name="c", subcore_axis_name="s")

    @functools.partial(
        pl.kernel,
        out_type=jax.ShapeDtypeStruct((ne, DP), _F32),
        mesh=mesh,
        scratch_types=(
            [pltpu.VMEM((GCH,), jnp.int32)] * NBUF
            + [pltpu.VMEM((GCH, DP), _F32)] * NBUF
            + [pltpu.SemaphoreType.DMA((NBUF,))] * 3
        ),
    )
    def k(table_hbm, src_hbm, out_hbm, i0, i1, i2, i3, r0, r1, r2, r3,
          isem, gsem, osem):
        idx_v = (i0, i1, i2, i3)
        rows_v = (r0, r1, r2, r3)
        wid = lax.axis_index("s") * NC + lax.axis_index("c")
        base = wid * epw

        def icp(ci):
            b = ci % NBUF
            return pltpu.make_async_copy(
                src_hbm.at[pl.ds(base + ci * GCH, GCH)], idx_v[b],
                isem.at[b])

        def gat(ci):
            b = ci % NBUF
            return pltpu.make_async_copy(
                table_hbm.at[idx_v[b]], rows_v[b], gsem.at[b])

        def wb(ci):
            b = ci % NBUF
            return pltpu.make_async_copy(
                rows_v[b], out_hbm.at[pl.ds(base + ci * GCH, GCH)],
                osem.at[b])

        icp(0).start()
        icp(1).start()
        for ci in range(nch):
            icp(ci).wait()
            gat(ci).start()
            if ci >= 1:
                gat(ci - 1).wait()
                wb(ci - 1).start()
            if ci >= 2:
                wb(ci - 2).wait()
            if ci + 2 < nch:
                icp(ci + 2).start()
        gat(nch - 1).wait()
        wb(nch - 1).start()
        wb(nch - 2).wait()
        wb(nch - 1).wait()

    return k(table, src)


def _sc_scatter(msg, dst, zeros_t, ne):
    epw = ne // NW
    chunks = _schunks(epw)
    nsch = len(chunks)
    mesh = plsc.VectorSubcoreMesh(core_axis_name="c", subcore_axis_name="s")
    ZR = NP // NS

    @functools.partial(
        pl.kernel,
        out_type=jax.ShapeDtypeStruct((NC, NP, DP), _F32),
        mesh=mesh,
        scratch_types=(
            [pltpu.VMEM((SCH,), jnp.int32)] * 2
            + [pltpu.VMEM((SCH, DP), _F32)] * 2
            + [pltpu.VMEM_SHARED((NP, DP), _F32)]
            + [pltpu.SemaphoreType.DMA((2,))] * 2
        ),
    )
    def k(msg_hbm, dst_hbm, zeros_hbm, agg_hbm, i0, i1, r0, r1,
          table_sh, lsem, ssem):
        idx_v = (i0, i1)
        rows_v = (r0, r1)
        c = lax.axis_index("c")
        s = lax.axis_index("s")
        pltpu.sync_copy(zeros_hbm.at[pl.ds(s * ZR, ZR)],
                        table_sh.at[pl.ds(s * ZR, ZR)])
        plsc.subcore_barrier()
        base = (s * NC + c) * epw

        def icp(ci):
            off, n, b = chunks[ci][0], chunks[ci][1], ci % 2
            return pltpu.make_async_copy(
                dst_hbm.at[pl.ds(base + off, n)],
                idx_v[b].at[pl.ds(0, n)], lsem.at[b])

        def mcp(ci):
            off, n, b = chunks[ci][0], chunks[ci][1], ci % 2
            return pltpu.make_async_copy(
                msg_hbm.at[pl.ds(base + off, n)],
                rows_v[b].at[pl.ds(0, n)], lsem.at[b])

        def sca(ci, start):
            n, b = chunks[ci][1], ci % 2
            d = pltpu.make_async_copy(
                rows_v[b].at[pl.ds(0, n)],
                table_sh.at[idx_v[b].at[pl.ds(0, n)]], ssem.at[b])
            if start:
                pltpu.async_copy(
                    rows_v[b].at[pl.ds(0, n)],
                    table_sh.at[idx_v[b].at[pl.ds(0, n)]], ssem.at[b],
                    add=True)
            else:
                d.wait()

        icp(0).start()
        mcp(0).start()
        for ci in range(nsch):
            icp(ci).wait()
            mcp(ci).wait()
            sca(ci, True)
            if ci >= 1:
                sca(ci - 1, False)
            if ci + 1 < nsch:
                icp(ci + 1).start()
                mcp(ci + 1).start()
        sca(nsch - 1, False)
        plsc.subcore_barrier()
        pltpu.sync_copy(table_sh.at[pl.ds(s * ZR, ZR)],
                        agg_hbm.at[c, pl.ds(s * ZR, ZR)])

    return k(msg, dst, zeros_t)


def _upd_body(aggpa_ref, aggpb_ref, out_ref, cb_ref, wm1_ref, wm2_ref,
              bm_ref, new_ref):
    agg = (aggpa_ref[0] + aggpa_ref[1]) + (aggpb_ref[0] + aggpb_ref[1])
    o = out_ref[...]
    m = jnp.maximum(agg + o + cb_ref[...], 0.0)
    y = (jnp.dot(m, wm1_ref[...], preferred_element_type=_F32)
         + jnp.dot(o, wm2_ref[...], preferred_element_type=_F32))
    new_ref[...] = y + bm_ref[...]


def _upd_res_body(aggpa_ref, aggpb_ref, out_ref, cb_ref, wm1_ref, wm2_ref,
                  bm_ref, nf_ref, new_ref):
    agg = (aggpa_ref[0] + aggpa_ref[1]) + (aggpb_ref[0] + aggpb_ref[1])
    o = out_ref[...]
    m = jnp.maximum(agg + o + cb_ref[...], 0.0)
    y = (jnp.dot(m, wm1_ref[...], preferred_element_type=_F32)
         + jnp.dot(o, wm2_ref[...], preferred_element_type=_F32))
    new_ref[...] = y + bm_ref[...] + nf_ref[...]


def _upd(aggpa, aggpb, out, cbp, wm1p, wm2p, bmp):
    return pl.pallas_call(
        _upd_body,
        out_shape=jax.ShapeDtypeStruct((NP, DP), _F32),
    )(aggpa, aggpb, out, cbp, wm1p, wm2p, bmp)


def _upd_res(aggpa, aggpb, out, cbp, wm1p, wm2p, bmp, nf_p):
    return pl.pallas_call(
        _upd_res_body,
        out_shape=jax.ShapeDtypeStruct((NP, DP), _F32),
    )(aggpa, aggpb, out, cbp, wm1p, wm2p, bmp, nf_p)


def _pad2(a, rows, cols):
    return jnp.pad(a, ((0, rows - a.shape[0]), (0, cols - a.shape[1])))


def kernel(n_feat, e_feat, edge_index, W0, b0, We1, be1, We2, be2,
           conv_bias, Wm, bm):
    src_a, src_b = edge_index[0, :NEA], edge_index[0, NEA:]
    dst_a, dst_b = edge_index[1, :NEA], edge_index[1, NEA:]

    nf_p = _pad2(n_feat, NP, DP)
    ef_p = _pad2(e_feat, NE, 16)
    w0p = _pad2(W0, DP, DP)
    b0p = _pad2(b0[None, :], 1, DP)
    we1p = _pad2(We1, 16, DH)
    be1r = be1[None, :]
    we2b = jnp.pad(We2.reshape(DH, DN, DN), ((0, 0), (0, 0), (0, OP - DN))
                   ).reshape(DH, FW).astype(jnp.bfloat16)
    be2b = jnp.pad(be2.reshape(1, DN, DN), ((0, 0), (0, 0), (0, OP - DN))
                   ).reshape(1, FW)
    cbp = _pad2(conv_bias[None, :], 1, DP)
    wm1p = _pad2(Wm[:DN], DP, DP)
    wm2p = _pad2(Wm[DN:], DP, DP)
    bmp = _pad2(bm[None, :], 1, DP)
    zeros_t = jnp.zeros((NP, DP), _F32)
    bsel = (jnp.arange(FW)[None, :] // OP ==
            jnp.arange(48)[:, None]).astype(jnp.bfloat16)

    out = _lin0(nf_p, w0p, b0p)
    ew_a = _wprep(ef_p[:NEA], we1p, be1r, we2b, be2b, NEA)
    ew_b = _wprep(ef_p[NEA:], we1p, be1r, we2b, be2b, NEB)

    for step in range(NSTEPS):
        h_a = _sc_gather(out, src_a, NEA)
        msg_a = _msg(ew_a, h_a, bsel, NEA)
        h_b = _sc_gather(out, src_b, NEB)
        msg_b = _msg(ew_b, h_b, bsel, NEB)
        aggp_a = _sc_scatter(msg_a, dst_a, zeros_t, NEA)
        aggp_b = _sc_scatter(msg_b, dst_b, zeros_t, NEB)
        if step < NSTEPS - 1:
            out = _upd(aggp_a, aggp_b, out, cbp, wm1p, wm2p, bmp)
        else:
            out = _upd_res(aggp_a, aggp_b, out, cbp, wm1p, wm2p, bmp, nf_p)

    return out[:NN, :DN]
